# baseline (speedup 1.0000x reference)
import jax
import jax.numpy as jnp
from jax.experimental import pallas as pl
from jax.experimental.pallas import tpu as pltpu

B = 4096
D = 1024
ED = 8
VOCAB = 100
SORT = 15
BLK = 512


def _dot(a, b):
    return jax.lax.dot_general(
        a, b, (((1,), (0,)), ((), ())),
        preferred_element_type=jnp.float32)


def _body(xs_ref, noise_ref, table_ref, w_sort_ref, b_sort_ref,
          w_note_ref, b_note_ref, note_ref, sort_ref, prob_ref):
    xs = xs_ref[...].astype(jnp.bfloat16)
    parent = table_ref[1:2, :]

    const_sort = _dot(parent, w_sort_ref[:ED, :]) + b_sort_ref[...]
    logits = _dot(xs, w_sort_ref[2 * ED:, :]) + const_sort

    m = jnp.max(logits, axis=-1, keepdims=True)
    e = jnp.exp(logits - m)
    p = e / jnp.sum(e, axis=-1, keepdims=True)
    prob_ref[...] = p

    gumbel = -jnp.log(-jnp.log(noise_ref[...]))
    scores = jnp.log(p) + gumbel
    smax = jnp.max(scores, axis=-1, keepdims=True)
    lane = jax.lax.broadcasted_iota(jnp.int32, (BLK, SORT), 1)
    idx = jnp.min(jnp.where(scores == smax, lane, SORT), axis=-1,
                  keepdims=True)
    sort_ref[...] = idx

    onehot = (jax.lax.broadcasted_iota(jnp.int32, (BLK, VOCAB), 1)
              == idx).astype(jnp.bfloat16)
    emb = _dot(onehot, table_ref[...]).astype(jnp.bfloat16)

    xe = jnp.concatenate([xs, emb], axis=1)
    const_note = _dot(parent, w_note_ref[:ED, :]) + b_note_ref[...]
    note_ref[...] = _dot(xe, w_note_ref[2 * ED:, :]) + const_note


def kernel(xs, noise, table, W_mask, b_mask, W_sort, b_sort, W_note, b_note):
    del W_mask, b_mask
    grid = (B // BLK,)
    note, sort2d, prob = pl.pallas_call(
        _body,
        grid=grid,
        in_specs=[
            pl.BlockSpec((BLK, D), lambda i: (i, 0)),
            pl.BlockSpec((BLK, SORT), lambda i: (i, 0)),
            pl.BlockSpec((VOCAB, ED), lambda i: (0, 0)),
            pl.BlockSpec((2 * ED + D, SORT), lambda i: (0, 0)),
            pl.BlockSpec((1, SORT), lambda i: (0, 0)),
            pl.BlockSpec((2 * ED + D + ED, D), lambda i: (0, 0)),
            pl.BlockSpec((1, D), lambda i: (0, 0)),
        ],
        out_specs=[
            pl.BlockSpec((BLK, D), lambda i: (i, 0)),
            pl.BlockSpec((BLK, 1), lambda i: (i, 0)),
            pl.BlockSpec((BLK, SORT), lambda i: (i, 0)),
        ],
        out_shape=[
            jax.ShapeDtypeStruct((B, D), jnp.float32),
            jax.ShapeDtypeStruct((B, 1), jnp.int32),
            jax.ShapeDtypeStruct((B, SORT), jnp.float32),
        ],
        compiler_params=pltpu.CompilerParams(
            dimension_semantics=("parallel",)),
    )(xs, noise, table.astype(jnp.bfloat16),
      W_sort.astype(jnp.bfloat16), b_sort.reshape(1, SORT),
      W_note.astype(jnp.bfloat16), b_note.reshape(1, D))
    return note, sort2d.reshape(B), prob


# speedup vs baseline: 1.0872x; 1.0872x over previous
import jax
import jax.numpy as jnp
from jax.experimental import pallas as pl
from jax.experimental.pallas import tpu as pltpu

B = 4096
D = 1024
ED = 8
VOCAB = 100
SORT = 15
BLK = 1024


def _dot(a, b):
    return jax.lax.dot_general(
        a, b, (((1,), (0,)), ((), ())),
        preferred_element_type=jnp.float32)


def _body(xs_ref, noise_ref, table_ref, w_sort_ref, b_sort_ref,
          w_note_ref, b_note_ref, note_ref, sort_ref, prob_ref,
          wn_bf, ws_bf):
    @pl.when(pl.program_id(0) == 0)
    def _init():
        wn_bf[...] = w_note_ref[...].astype(jnp.bfloat16)
        ws_bf[...] = w_sort_ref[...].astype(jnp.bfloat16)

    xs = xs_ref[...].astype(jnp.bfloat16)
    parent = table_ref[1:2, :].astype(jnp.bfloat16)

    const_sort = _dot(parent, ws_bf[:ED, :]) + b_sort_ref[...]
    logits = _dot(xs, ws_bf[2 * ED:, :]) + const_sort

    m = jnp.max(logits, axis=-1, keepdims=True)
    e = jnp.exp(logits - m)
    p = e / jnp.sum(e, axis=-1, keepdims=True)
    prob_ref[...] = p

    gumbel = -jnp.log(-jnp.log(noise_ref[...]))
    scores = jnp.log(p) + gumbel
    smax = jnp.max(scores, axis=-1, keepdims=True)
    lane = jax.lax.broadcasted_iota(jnp.int32, (BLK, SORT), 1)
    idx = jnp.min(jnp.where(scores == smax, lane, SORT), axis=-1,
                  keepdims=True)
    sort_ref[...] = idx

    onehot = (jax.lax.broadcasted_iota(jnp.int32, (BLK, VOCAB), 1)
              == idx).astype(jnp.bfloat16)
    emb = _dot(onehot, table_ref[...].astype(jnp.bfloat16))
    emb = emb.astype(jnp.bfloat16)

    xe = jnp.concatenate([xs, emb], axis=1)
    const_note = _dot(parent, wn_bf[:ED, :]) + b_note_ref[...]
    note_ref[...] = _dot(xe, wn_bf[2 * ED:, :]) + const_note


def kernel(xs, noise, table, W_mask, b_mask, W_sort, b_sort, W_note, b_note):
    del W_mask, b_mask
    grid = (B // BLK,)
    note, sort2d, prob = pl.pallas_call(
        _body,
        grid=grid,
        in_specs=[
            pl.BlockSpec((BLK, D), lambda i: (i, 0)),
            pl.BlockSpec((BLK, SORT), lambda i: (i, 0)),
            pl.BlockSpec((VOCAB, ED), lambda i: (0, 0)),
            pl.BlockSpec((2 * ED + D, SORT), lambda i: (0, 0)),
            pl.BlockSpec((1, SORT), lambda i: (0, 0)),
            pl.BlockSpec((2 * ED + D + ED, D), lambda i: (0, 0)),
            pl.BlockSpec((1, D), lambda i: (0, 0)),
        ],
        out_specs=[
            pl.BlockSpec((BLK, D), lambda i: (i, 0)),
            pl.BlockSpec((BLK, 1), lambda i: (i, 0)),
            pl.BlockSpec((BLK, SORT), lambda i: (i, 0)),
        ],
        out_shape=[
            jax.ShapeDtypeStruct((B, D), jnp.float32),
            jax.ShapeDtypeStruct((B, 1), jnp.int32),
            jax.ShapeDtypeStruct((B, SORT), jnp.float32),
        ],
        scratch_shapes=[
            pltpu.VMEM((2 * ED + D + ED, D), jnp.bfloat16),
            pltpu.VMEM((2 * ED + D, SORT), jnp.bfloat16),
        ],
        compiler_params=pltpu.CompilerParams(
            dimension_semantics=("arbitrary",)),
    )(xs, noise, table, W_sort, b_sort.reshape(1, SORT),
      W_note, b_note.reshape(1, D))
    return note, sort2d.reshape(B), prob
